# Initial kernel scaffold; baseline (speedup 1.0000x reference)
import jax
import jax.numpy as jnp
from jax.experimental import pallas as pl


def kernel(x, idx, W1, b1, g1, be1, W2, b2, g2, be2):
    raise NotImplementedError("write your pallas kernel here")


# speedup vs baseline: 6.6273x; 6.6273x over previous
import functools

import jax
import jax.numpy as jnp
from jax import lax
from jax.experimental import pallas as pl
from jax.experimental.pallas import tpu as pltpu
from jax.experimental.pallas import tpu_sc as plsc

EPS = 1e-5


def _prep_body(x_ref, wu_ref, wv_ref, b1_ref, u_ref, v_ref):
    xb = x_ref[...]
    dn = (((0,), (0,)), ((), ()))
    u = lax.dot_general(xb, wu_ref[...], dn,
                        preferred_element_type=jnp.float32,
                        precision=lax.Precision.HIGHEST)
    v = lax.dot_general(xb, wv_ref[...], dn,
                        preferred_element_type=jnp.float32,
                        precision=lax.Precision.HIGHEST)
    u_ref[...] = u + b1_ref[...]
    v_ref[...] = v


def _tc_prep(x2, wu_t, wv_t, b1, nb=10000):
    c, n = x2.shape
    out = x2.shape[1]
    grid = n // nb
    return pl.pallas_call(
        _prep_body,
        grid=(grid,),
        in_specs=[
            pl.BlockSpec((c, nb), lambda i: (0, i)),
            pl.BlockSpec((c, wu_t.shape[1]), lambda i: (0, 0)),
            pl.BlockSpec((c, wv_t.shape[1]), lambda i: (0, 0)),
            pl.BlockSpec((1, wu_t.shape[1]), lambda i: (0, 0)),
        ],
        out_specs=[
            pl.BlockSpec((nb, wu_t.shape[1]), lambda i: (i, 0)),
            pl.BlockSpec((nb, wv_t.shape[1]), lambda i: (i, 0)),
        ],
        out_shape=[
            jax.ShapeDtypeStruct((n, wu_t.shape[1]), jnp.float32),
            jax.ShapeDtypeStruct((n, wv_t.shape[1]), jnp.float32),
        ],
    )(x2, wu_t, wv_t, b1)


_SC_CORES = 2
_SC_SUBCORES = 16
_NW = _SC_CORES * _SC_SUBCORES


def _sc_gather(v_t, idx_flat):
    n_edges = idx_flat.shape[0]
    d = v_t.shape[1]
    per_w = n_edges // _NW
    ch = 40
    n_ch = per_w // ch
    mesh = plsc.VectorSubcoreMesh(core_axis_name="c", subcore_axis_name="s")

    @functools.partial(
        pl.kernel,
        mesh=mesh,
        out_type=jax.ShapeDtypeStruct((n_edges, d), jnp.float32),
        scratch_types=[
            pltpu.VMEM((per_w,), jnp.int32),
            pltpu.VMEM((ch, d), jnp.float32),
            pltpu.VMEM((ch, d), jnp.float32),
            pltpu.SemaphoreType.DMA,
            pltpu.SemaphoreType.DMA,
            pltpu.SemaphoreType.DMA,
        ],
    )
    def gather_kernel(table_hbm, idx_hbm, out_hbm, idx_all, buf0, buf1,
                      sem0, sem1, semi):
        wid = lax.axis_index("s") * _SC_CORES + lax.axis_index("c")
        base = wid * per_w
        pltpu.async_copy(idx_hbm.at[pl.ds(base, per_w)], idx_all, semi).wait()

        def gat(c, buf, sem):
            return pltpu.make_async_copy(
                table_hbm.at[idx_all.at[pl.ds(c * ch, ch)]], buf, sem)

        gat(0, buf0, sem0).start()

        @pl.loop(0, n_ch // 2)
        def _(p):
            c0 = p * 2
            c1 = c0 + 1
            gat(c0, buf0, sem0).wait()
            gat(c1, buf1, sem1).start()
            pltpu.sync_copy(buf0, out_hbm.at[pl.ds(base + c0 * ch, ch)])
            gat(c1, buf1, sem1).wait()

            @pl.when(p < n_ch // 2 - 1)
            def _():
                gat(c0 + 2, buf0, sem0).start()

            pltpu.sync_copy(buf1, out_hbm.at[pl.ds(base + c1 * ch, ch)])

    return gather_kernel(v_t, idx_flat)


def _stats1_body(y_ref, u_ref, s_ref, *, nb, k):
    i = pl.program_id(0)

    @pl.when(i == 0)
    def _():
        s_ref[...] = jnp.zeros_like(s_ref)

    d = y_ref.shape[-1]
    h = y_ref[...].reshape(nb, k, d) + u_ref[...][:, None, :]
    s1 = jnp.sum(h, axis=(0, 1))
    s2 = jnp.sum(h * h, axis=(0, 1))
    s_ref[...] += jnp.stack([s1, s2], axis=0)


def _tc_stats1(y0, u_t, k, nb=400):
    n, d = u_t.shape
    grid = n // nb
    eb = nb * k
    return pl.pallas_call(
        functools.partial(_stats1_body, nb=nb, k=k),
        grid=(grid,),
        in_specs=[
            pl.BlockSpec((eb, d), lambda i: (i, 0)),
            pl.BlockSpec((nb, d), lambda i: (i, 0)),
        ],
        out_specs=pl.BlockSpec((2, d), lambda i: (0, 0)),
        out_shape=jax.ShapeDtypeStruct((2, d), jnp.float32),
    )(y0, u_t)


def _main_body(y_ref, u_ref, a1_ref, c1_ref, w2t_ref, b2_ref,
               mx_ref, mn_ref, s_ref, *, nb, k):
    i = pl.program_id(0)

    @pl.when(i == 0)
    def _():
        s_ref[...] = jnp.zeros_like(s_ref)

    d = y_ref.shape[-1]
    h1 = y_ref[...].reshape(nb, k, d) + u_ref[...][:, None, :]
    y = jnp.maximum(h1 * a1_ref[...][:, None, :] + c1_ref[...][:, None, :],
                    0.0)
    h2 = lax.dot_general(y.reshape(nb * k, d), w2t_ref[...],
                         (((1,), (0,)), ((), ())),
                         preferred_element_type=jnp.float32,
                         precision=lax.Precision.HIGHEST)
    h2 = h2 + b2_ref[...]
    s1 = jnp.sum(h2, axis=0)
    s2 = jnp.sum(h2 * h2, axis=0)
    s_ref[...] += jnp.stack([s1, s2], axis=0)
    h23 = h2.reshape(nb, k, d)
    mx_ref[...] = jnp.max(h23, axis=1)
    mn_ref[...] = jnp.min(h23, axis=1)


def _tc_main(y0, u_t, a1, c1, w2t, b2, k, nb=400):
    n, d = u_t.shape
    grid = n // nb
    eb = nb * k
    return pl.pallas_call(
        functools.partial(_main_body, nb=nb, k=k),
        grid=(grid,),
        in_specs=[
            pl.BlockSpec((eb, d), lambda i: (i, 0)),
            pl.BlockSpec((nb, d), lambda i: (i, 0)),
            pl.BlockSpec((1, d), lambda i: (0, 0)),
            pl.BlockSpec((1, d), lambda i: (0, 0)),
            pl.BlockSpec((d, d), lambda i: (0, 0)),
            pl.BlockSpec((1, d), lambda i: (0, 0)),
        ],
        out_specs=[
            pl.BlockSpec((nb, d), lambda i: (i, 0)),
            pl.BlockSpec((nb, d), lambda i: (i, 0)),
            pl.BlockSpec((2, d), lambda i: (0, 0)),
        ],
        out_shape=[
            jax.ShapeDtypeStruct((n, d), jnp.float32),
            jax.ShapeDtypeStruct((n, d), jnp.float32),
            jax.ShapeDtypeStruct((2, d), jnp.float32),
        ],
    )(y0, u_t, a1, c1, w2t, b2)


def _final_body(mx_ref, mn_ref, a2_ref, c2_ref, o_ref):
    a2 = a2_ref[...]
    m = jnp.where(a2 >= 0.0, mx_ref[...], mn_ref[...])
    r = jnp.maximum(a2 * m + c2_ref[...], 0.0)
    o_ref[...] = r.T


def _tc_final(mx, mn, a2, c2, nb=10000):
    n, d = mx.shape
    grid = n // nb
    return pl.pallas_call(
        _final_body,
        grid=(grid,),
        in_specs=[
            pl.BlockSpec((nb, d), lambda i: (i, 0)),
            pl.BlockSpec((nb, d), lambda i: (i, 0)),
            pl.BlockSpec((1, d), lambda i: (0, 0)),
            pl.BlockSpec((1, d), lambda i: (0, 0)),
        ],
        out_specs=pl.BlockSpec((d, nb), lambda i: (0, i)),
        out_shape=jax.ShapeDtypeStruct((d, n), jnp.float32),
    )(mx, mn, a2, c2)


def _bn_coeffs(stats, gamma, beta, count):
    mean = stats[0] / count
    var = stats[1] / count - mean * mean
    a = gamma * lax.rsqrt(var + EPS)
    c = beta - a * mean
    return a[None, :], c[None, :]


@jax.jit
def kernel(x, idx, W1, b1, g1, be1, W2, b2, g2, be2):
    b, c, n = x.shape
    k = idx.shape[-1]
    out_ch = W1.shape[0]

    x2 = x[0]
    w1a = W1[:, :c]
    w1b = W1[:, c:]
    wu_t = (w1a - w1b).T
    wv_t = w1b.T

    u_t, v_t = _tc_prep(x2, wu_t, wv_t, b1[None, :])

    idx_flat = idx.reshape(-1)
    y0 = _sc_gather(v_t, idx_flat)

    count = jnp.float32(b * n * k)
    stats1 = _tc_stats1(y0, u_t, k)
    a1, c1 = _bn_coeffs(stats1, g1, be1, count)

    mx, mn, stats2 = _tc_main(y0, u_t, a1, c1, W2.T, b2[None, :], k)
    a2, c2 = _bn_coeffs(stats2, g2, be2, count)

    out = _tc_final(mx, mn, a2, c2)
    return out[None]


# speedup vs baseline: 7.9751x; 1.2034x over previous
import functools

import jax
import jax.numpy as jnp
from jax import lax
from jax.experimental import pallas as pl
from jax.experimental.pallas import tpu as pltpu
from jax.experimental.pallas import tpu_sc as plsc

EPS = 1e-5


def _prep_body(x_ref, wu_ref, wv_ref, b1_ref, u_ref, v_ref):
    xb = x_ref[...]
    dn = (((0,), (0,)), ((), ()))
    u = lax.dot_general(xb, wu_ref[...], dn,
                        preferred_element_type=jnp.float32,
                        precision=lax.Precision.HIGHEST)
    v = lax.dot_general(xb, wv_ref[...], dn,
                        preferred_element_type=jnp.float32,
                        precision=lax.Precision.HIGHEST)
    u_ref[...] = u + b1_ref[...]
    v_ref[...] = v


def _tc_prep(x2, wu_t, wv_t, b1, nb=10000):
    c, n = x2.shape
    out = x2.shape[1]
    grid = n // nb
    return pl.pallas_call(
        _prep_body,
        grid=(grid,),
        in_specs=[
            pl.BlockSpec((c, nb), lambda i: (0, i)),
            pl.BlockSpec((c, wu_t.shape[1]), lambda i: (0, 0)),
            pl.BlockSpec((c, wv_t.shape[1]), lambda i: (0, 0)),
            pl.BlockSpec((1, wu_t.shape[1]), lambda i: (0, 0)),
        ],
        out_specs=[
            pl.BlockSpec((nb, wu_t.shape[1]), lambda i: (i, 0)),
            pl.BlockSpec((nb, wv_t.shape[1]), lambda i: (i, 0)),
        ],
        out_shape=[
            jax.ShapeDtypeStruct((n, wu_t.shape[1]), jnp.float32),
            jax.ShapeDtypeStruct((n, wv_t.shape[1]), jnp.float32),
        ],
    )(x2, wu_t, wv_t, b1)


_SC_CORES = 2
_SC_SUBCORES = 16
_NW = _SC_CORES * _SC_SUBCORES


def _sc_gather(v_t, idx_flat):
    n_edges = idx_flat.shape[0]
    d = v_t.shape[1]
    per_w = n_edges // _NW
    ch = 40
    n_ch = per_w // ch
    mesh = plsc.VectorSubcoreMesh(core_axis_name="c", subcore_axis_name="s")

    @functools.partial(
        pl.kernel,
        mesh=mesh,
        out_type=jax.ShapeDtypeStruct((n_edges, d), jnp.float32),
        scratch_types=[
            pltpu.VMEM((per_w,), jnp.int32),
            pltpu.VMEM((ch, d), jnp.float32),
            pltpu.VMEM((ch, d), jnp.float32),
            pltpu.SemaphoreType.DMA,
            pltpu.SemaphoreType.DMA,
            pltpu.SemaphoreType.DMA,
        ],
    )
    def gather_kernel(table_hbm, idx_hbm, out_hbm, idx_all, buf0, buf1,
                      sem0, sem1, semi):
        wid = lax.axis_index("s") * _SC_CORES + lax.axis_index("c")
        base = wid * per_w
        pltpu.async_copy(idx_hbm.at[pl.ds(base, per_w)], idx_all, semi).wait()

        def gat(c, buf, sem):
            return pltpu.make_async_copy(
                table_hbm.at[idx_all.at[pl.ds(c * ch, ch)]], buf, sem)

        gat(0, buf0, sem0).start()

        @pl.loop(0, n_ch // 2)
        def _(p):
            c0 = p * 2
            c1 = c0 + 1
            gat(c0, buf0, sem0).wait()
            gat(c1, buf1, sem1).start()
            pltpu.sync_copy(buf0, out_hbm.at[pl.ds(base + c0 * ch, ch)])
            gat(c1, buf1, sem1).wait()

            @pl.when(p < n_ch // 2 - 1)
            def _():
                gat(c0 + 2, buf0, sem0).start()

            pltpu.sync_copy(buf1, out_hbm.at[pl.ds(base + c1 * ch, ch)])

    return gather_kernel(v_t, idx_flat)


def _stats1_body(y_ref, u_ref, s_ref, *, nb, k):
    i = pl.program_id(0)

    @pl.when(i == 0)
    def _():
        s_ref[...] = jnp.zeros_like(s_ref)

    d = y_ref.shape[-1]
    h = y_ref[...].reshape(nb, k, d) + u_ref[...][:, None, :]
    s1 = jnp.sum(h, axis=(0, 1))
    s2 = jnp.sum(h * h, axis=(0, 1))
    s_ref[...] += jnp.stack([s1, s2], axis=0)


def _tc_stats1(y0, u_t, k, nb=400):
    n, d = u_t.shape
    grid = n // nb
    eb = nb * k
    return pl.pallas_call(
        functools.partial(_stats1_body, nb=nb, k=k),
        grid=(grid,),
        in_specs=[
            pl.BlockSpec((eb, d), lambda i: (i, 0)),
            pl.BlockSpec((nb, d), lambda i: (i, 0)),
        ],
        out_specs=pl.BlockSpec((2, d), lambda i: (0, 0)),
        out_shape=jax.ShapeDtypeStruct((2, d), jnp.float32),
    )(y0, u_t)


def _main_body(y_ref, u_ref, a1_ref, c1_ref, w2t_ref, b2_ref,
               mx_ref, mn_ref, s_ref, *, nb, k):
    i = pl.program_id(0)

    @pl.when(i == 0)
    def _():
        s_ref[...] = jnp.zeros_like(s_ref)

    d = y_ref.shape[-1]
    h1 = y_ref[...].reshape(nb, k, d) + u_ref[...][:, None, :]
    y = jnp.maximum(h1 * a1_ref[...][:, None, :] + c1_ref[...][:, None, :],
                    0.0)
    h2 = lax.dot_general(y.reshape(nb * k, d), w2t_ref[...],
                         (((1,), (0,)), ((), ())),
                         preferred_element_type=jnp.float32,
                         precision=lax.Precision.DEFAULT)
    h2 = h2 + b2_ref[...]
    s1 = jnp.sum(h2, axis=0)
    s2 = jnp.sum(h2 * h2, axis=0)
    s_ref[...] += jnp.stack([s1, s2], axis=0)
    h23 = h2.reshape(nb, k, d)
    mx_ref[...] = jnp.max(h23, axis=1)
    mn_ref[...] = jnp.min(h23, axis=1)


def _tc_main(y0, u_t, a1, c1, w2t, b2, k, nb=400):
    n, d = u_t.shape
    grid = n // nb
    eb = nb * k
    return pl.pallas_call(
        functools.partial(_main_body, nb=nb, k=k),
        grid=(grid,),
        in_specs=[
            pl.BlockSpec((eb, d), lambda i: (i, 0)),
            pl.BlockSpec((nb, d), lambda i: (i, 0)),
            pl.BlockSpec((1, d), lambda i: (0, 0)),
            pl.BlockSpec((1, d), lambda i: (0, 0)),
            pl.BlockSpec((d, d), lambda i: (0, 0)),
            pl.BlockSpec((1, d), lambda i: (0, 0)),
        ],
        out_specs=[
            pl.BlockSpec((nb, d), lambda i: (i, 0)),
            pl.BlockSpec((nb, d), lambda i: (i, 0)),
            pl.BlockSpec((2, d), lambda i: (0, 0)),
        ],
        out_shape=[
            jax.ShapeDtypeStruct((n, d), jnp.float32),
            jax.ShapeDtypeStruct((n, d), jnp.float32),
            jax.ShapeDtypeStruct((2, d), jnp.float32),
        ],
    )(y0, u_t, a1, c1, w2t, b2)


def _final_body(mx_ref, mn_ref, a2_ref, c2_ref, o_ref):
    a2 = a2_ref[...]
    m = jnp.where(a2 >= 0.0, mx_ref[...], mn_ref[...])
    r = jnp.maximum(a2 * m + c2_ref[...], 0.0)
    o_ref[...] = r.T


def _tc_final(mx, mn, a2, c2, nb=10000):
    n, d = mx.shape
    grid = n // nb
    return pl.pallas_call(
        _final_body,
        grid=(grid,),
        in_specs=[
            pl.BlockSpec((nb, d), lambda i: (i, 0)),
            pl.BlockSpec((nb, d), lambda i: (i, 0)),
            pl.BlockSpec((1, d), lambda i: (0, 0)),
            pl.BlockSpec((1, d), lambda i: (0, 0)),
        ],
        out_specs=pl.BlockSpec((d, nb), lambda i: (0, i)),
        out_shape=jax.ShapeDtypeStruct((d, n), jnp.float32),
    )(mx, mn, a2, c2)


def _bn_coeffs(stats, gamma, beta, count):
    mean = stats[0] / count
    var = stats[1] / count - mean * mean
    a = gamma * lax.rsqrt(var + EPS)
    c = beta - a * mean
    return a[None, :], c[None, :]


@jax.jit
def kernel(x, idx, W1, b1, g1, be1, W2, b2, g2, be2):
    b, c, n = x.shape
    k = idx.shape[-1]
    out_ch = W1.shape[0]

    x2 = x[0]
    w1a = W1[:, :c]
    w1b = W1[:, c:]
    wu_t = (w1a - w1b).T
    wv_t = w1b.T

    u_t, v_t = _tc_prep(x2, wu_t, wv_t, b1[None, :])

    idx_flat = idx.reshape(-1)
    y0 = _sc_gather(v_t, idx_flat)

    count = jnp.float32(b * n * k)
    stats1 = _tc_stats1(y0, u_t, k)
    a1, c1 = _bn_coeffs(stats1, g1, be1, count)

    mx, mn, stats2 = _tc_main(y0, u_t, a1, c1, W2.T, b2[None, :], k)
    a2, c2 = _bn_coeffs(stats2, g2, be2, count)

    out = _tc_final(mx, mn, a2, c2)
    return out[None]
